# baseline (speedup 1.0000x reference)
import jax
import jax.numpy as jnp
from jax.experimental import pallas as pl
from jax.experimental.pallas import tpu as pltpu


def _final_body(pre_ref, o_ref):
    v = pre_ref[...]
    o_ref[...] = jnp.where(v >= 0, v, 0.01 * v)


def kernel(x, edge_index, sim_weight, rep, node_signal, W, W_self, alpha, beta, alpha_self, temp):
    row = edge_index[0]
    col = edge_index[1]
    h = x @ W.T
    h_j = jnp.take(h, col, axis=0)
    gate_input = alpha * jnp.take(rep, row) + beta * jnp.take(rep, col)
    gate = jax.nn.sigmoid(gate_input / temp)[:, None]
    sim_norm = jnp.zeros_like(sim_weight).at[row].add(sim_weight)
    sim = (sim_weight / (jnp.take(sim_norm, row) + 1e-06))[:, None]
    s_j = jnp.tanh(jnp.take(node_signal, col))[:, None]
    msg = sim * gate * s_j * h_j
    out = jnp.zeros_like(x).at[row].add(msg)
    deg = jnp.zeros((x.shape[0],), dtype=x.dtype).at[row].add(gate[:, 0])
    out = out / (deg[:, None] + 1e-06)
    gate_self = jax.nn.sigmoid(alpha_self * rep / temp)[:, None]
    pre = out + gate_self * (x @ W_self.T)
    n, d = pre.shape
    blk = 400
    return pl.pallas_call(
        _final_body,
        grid=(n // blk,),
        in_specs=[pl.BlockSpec((blk, d), lambda i: (i, 0))],
        out_specs=pl.BlockSpec((blk, d), lambda i: (i, 0)),
        out_shape=jax.ShapeDtypeStruct((n, d), pre.dtype),
    )(pre)


# speedup vs baseline: 6.8004x; 6.8004x over previous
import functools

import jax
import jax.numpy as jnp
from jax import lax
from jax.experimental import pallas as pl
from jax.experimental.pallas import tpu as pltpu
from jax.experimental.pallas import tpu_sc as plsc

N = 10000
E = 160000
DIM = 256
HD = DIM // 2
NC, NS, L = 2, 16, 16
NH = 5120
TR = NH
AR = 5248
RS = AR // NS
EPT = E // NS
C = 400
NCHUNK = EPT // C
NV = C // L
NVH = HD // L


def _sigmoid(t):
    return 1.0 / (1.0 + jnp.exp(-t))


def _mm_body(x_ref, wt_ref, o_ref):
    o_ref[...] = jnp.dot(x_ref[...], wt_ref[...],
                         preferred_element_type=jnp.float32)


def _final_body(x_ref, wt_ref, agg_ref, simn_ref, deg_ref, rs_ref, o_ref):
    hs = jnp.dot(x_ref[...], wt_ref[...], preferred_element_type=jnp.float32)
    den = (simn_ref[...] + 1e-6) * (deg_ref[...] + 1e-6)
    v = agg_ref[...] / den + _sigmoid(rs_ref[...]) * hs
    o_ref[...] = jnp.where(v >= 0, v, 0.01 * v)


def _sc_body(do_norm, row_hbm, col_hbm, w_hbm, ra_hbm, rb_hbm, sn_hbm, h_hbm,
             *rest):
    if do_norm:
        (agga_hbm, aggb_hbm, simna_hbm, simnb_hbm, dega_hbm, degb_hbm,
         rowc, colc, idxb, wcb, rab, rbb, snb, gb, coefb, msg, zb,
         acc_sh, simn_sh, deg_sh, sem) = rest
    else:
        (agga_hbm, aggb_hbm,
         rowc, colc, idxb, wcb, rab, rbb, snb, gb, coefb, msg, zb,
         acc_sh, sem) = rest
        simn_sh = deg_sh = None
    cid = lax.axis_index("c")
    sid = lax.axis_index("s")
    base = cid * NH

    zero16 = jnp.zeros((L,), jnp.float32)

    def _zmsg(i, _):
        for v in range(NVH):
            msg[i, pl.ds(v * L, L)] = zero16
        return _
    lax.fori_loop(0, RS, _zmsg, None)
    r0 = sid * RS
    pltpu.sync_copy(msg.at[pl.ds(0, RS)], acc_sh.at[pl.ds(r0, RS)])
    if do_norm:
        for j in range(RS // L):
            zb[pl.ds(j * L, L)] = zero16
        pltpu.sync_copy(zb, simn_sh.at[pl.ds(r0, RS)])
        pltpu.sync_copy(zb, deg_sh.at[pl.ds(r0, RS)])
    plsc.subcore_barrier()

    def _chunk(k, _):
        ebase = sid * EPT + k * C
        pltpu.sync_copy(row_hbm.at[pl.ds(ebase, C)], rowc)
        pltpu.sync_copy(col_hbm.at[pl.ds(ebase, C)], colc)
        pltpu.sync_copy(w_hbm.at[pl.ds(ebase, C)], wcb)
        cp1 = pltpu.async_copy(ra_hbm.at[rowc], rab, sem)
        cp2 = pltpu.async_copy(rb_hbm.at[colc], rbb, sem)
        cp3 = pltpu.async_copy(sn_hbm.at[colc], snb, sem)
        cp4 = pltpu.async_copy(h_hbm.at[colc], msg, sem)
        cp1.wait()
        cp2.wait()
        cp3.wait()
        cp4.wait()

        def _coef(v, _):
            sl = pl.ds(v * L, L)
            off = rowc[sl] - base
            valid = (off >= 0) & (off < NH)
            idxb[sl] = jnp.where(valid, off, TR)
            g = _sigmoid(rab[sl] + rbb[sl])
            z = snb[sl]
            s = 1.0 - 2.0 / (jnp.exp(2.0 * z) + 1.0)
            gb[sl] = g
            coefb[sl] = wcb[sl] * g * s
            return _
        lax.fori_loop(0, NV, _coef, None)

        def _edge(i, _):
            c1 = coefb[pl.ds(i, L)][0]
            for v in range(NVH):
                sl = pl.ds(v * L, L)
                msg[i, sl] = msg[i, sl] * c1
            return _
        lax.fori_loop(0, C, _edge, None)

        pltpu.sync_copy(msg, acc_sh.at[idxb], add=True)

        if do_norm:
            pltpu.sync_copy(gb.at[pl.ds(0, C)], deg_sh.at[idxb], add=True)
            pltpu.sync_copy(wcb, simn_sh.at[idxb], add=True)
        return _
    lax.fori_loop(0, NCHUNK, _chunk, None)

    plsc.subcore_barrier()

    @pl.when(cid == 0)
    def _():
        pltpu.sync_copy(acc_sh.at[pl.ds(r0, RS)], agga_hbm.at[pl.ds(r0, RS)])
        if do_norm:
            pltpu.sync_copy(simn_sh.at[pl.ds(r0, RS)], zb)
            pltpu.sync_copy(zb, simna_hbm.at[pl.ds(r0, RS)])
            pltpu.sync_copy(deg_sh.at[pl.ds(r0, RS)], zb)
            pltpu.sync_copy(zb, dega_hbm.at[pl.ds(r0, RS)])

    @pl.when(cid == 1)
    def _():
        pltpu.sync_copy(acc_sh.at[pl.ds(r0, RS)], aggb_hbm.at[pl.ds(r0, RS)])
        if do_norm:
            pltpu.sync_copy(simn_sh.at[pl.ds(r0, RS)], zb)
            pltpu.sync_copy(zb, simnb_hbm.at[pl.ds(r0, RS)])
            pltpu.sync_copy(deg_sh.at[pl.ds(r0, RS)], zb)
            pltpu.sync_copy(zb, degb_hbm.at[pl.ds(r0, RS)])


def _make_sc(do_norm):
    f32 = jnp.float32
    out_type = [jax.ShapeDtypeStruct((AR, HD), f32),
                jax.ShapeDtypeStruct((AR, HD), f32)]
    if do_norm:
        out_type += [jax.ShapeDtypeStruct((AR,), f32),
                     jax.ShapeDtypeStruct((AR,), f32),
                     jax.ShapeDtypeStruct((AR,), f32),
                     jax.ShapeDtypeStruct((AR,), f32)]
    scratch = [
        pltpu.VMEM((C,), jnp.int32),
        pltpu.VMEM((C,), jnp.int32),
        pltpu.VMEM((C,), jnp.int32),
        pltpu.VMEM((C,), f32),
        pltpu.VMEM((C,), f32),
        pltpu.VMEM((C,), f32),
        pltpu.VMEM((C,), f32),
        pltpu.VMEM((C + L,), f32),
        pltpu.VMEM((C + L,), f32),
        pltpu.VMEM((C, HD), f32),
        pltpu.VMEM((RS,), f32),
        pltpu.VMEM_SHARED((AR, HD), f32),
    ]
    if do_norm:
        scratch += [pltpu.VMEM_SHARED((AR,), f32),
                    pltpu.VMEM_SHARED((AR,), f32)]
    scratch += [pltpu.SemaphoreType.DMA]
    mesh = plsc.VectorSubcoreMesh(core_axis_name="c", subcore_axis_name="s",
                                  num_cores=NC, num_subcores=NS)
    return pl.kernel(functools.partial(_sc_body, do_norm),
                     out_type=out_type, mesh=mesh, scratch_types=scratch)


def kernel(x, edge_index, sim_weight, rep, node_signal, W, W_self,
           alpha, beta, alpha_self, temp):
    f32 = jnp.float32
    row = edge_index[0].astype(jnp.int32)
    col = edge_index[1].astype(jnp.int32)
    sim_weight = sim_weight.astype(f32)

    mm = pl.pallas_call(
        _mm_body,
        grid=(10,),
        in_specs=[pl.BlockSpec((N // 10, DIM), lambda i: (i, 0)),
                  pl.BlockSpec((DIM, DIM), lambda i: (0, 0))],
        out_specs=pl.BlockSpec((N // 10, DIM), lambda i: (i, 0)),
        out_shape=jax.ShapeDtypeStruct((N, DIM), f32),
    )
    h = mm(x, W.T)

    ra = (alpha / temp) * rep
    rb = (beta / temp) * rep
    rs = ((alpha_self / temp) * rep)[:, None]

    a0_lo, a0_hi, simn_lo, simn_hi, deg_lo, deg_hi = _make_sc(True)(
        row, col, sim_weight, ra, rb, node_signal, h[:, :HD])
    a1_lo, a1_hi = _make_sc(False)(
        row, col, sim_weight, ra, rb, node_signal, h[:, HD:])

    agg = jnp.concatenate(
        [jnp.concatenate([a0_lo[:NH], a1_lo[:NH]], axis=1),
         jnp.concatenate([a0_hi[:N - NH], a1_hi[:N - NH]], axis=1)], axis=0)
    simn = jnp.concatenate([simn_lo[:NH], simn_hi[:N - NH]])[:, None]
    deg = jnp.concatenate([deg_lo[:NH], deg_hi[:N - NH]])[:, None]

    fin = pl.pallas_call(
        _final_body,
        grid=(10,),
        in_specs=[pl.BlockSpec((N // 10, DIM), lambda i: (i, 0)),
                  pl.BlockSpec((DIM, DIM), lambda i: (0, 0)),
                  pl.BlockSpec((N // 10, DIM), lambda i: (i, 0)),
                  pl.BlockSpec((N // 10, 1), lambda i: (i, 0)),
                  pl.BlockSpec((N // 10, 1), lambda i: (i, 0)),
                  pl.BlockSpec((N // 10, 1), lambda i: (i, 0))],
        out_specs=pl.BlockSpec((N // 10, DIM), lambda i: (i, 0)),
        out_shape=jax.ShapeDtypeStruct((N, DIM), f32),
    )
    return fin(x, W_self.T, agg, simn, deg, rs)
